# baseline (speedup 1.0000x reference)
import functools
import math

import jax
import jax.numpy as jnp
from jax import lax
from jax.experimental import pallas as pl
from jax.experimental.pallas import tpu as pltpu
from jax.experimental.pallas import tpu_sc as plsc

N = 10000
D = 128
OUT = 128
HEADS = 8
DK = OUT // HEADS
E = 160000
NC = 2
NS = 16
NW = NC * NS
CHUNK = 128
NCHUNK = E // CHUNK
SLAB = 640


def _per_sub_rows(sid, copy_fn):
    @pl.when(sid < NS - 1)
    def _():
        copy_fn(pl.multiple_of(sid * SLAB, SLAB), SLAB)

    @pl.when(sid == NS - 1)
    def _():
        copy_fn((NS - 1) * SLAB, N - (NS - 1) * SLAB)
@functools.cache
def _mesh():
    return plsc.VectorSubcoreMesh(core_axis_name="c", subcore_axis_name="s",
                                  num_cores=NC, num_subcores=NS)

f32 = jnp.float32


def _dense_hk_body(x0, x1, w0, b0, w1, b1, hk0, hk1):
    hk0[...] = jnp.dot(x0[...], w0[...], preferred_element_type=f32) + b0[...]
    hk1[...] = jnp.dot(x1[...], w1[...], preferred_element_type=f32) + b1[...]


def _dense_hk(x0, x1, w0, b0, w1, b1):
    blk = 2000
    io = pl.BlockSpec((blk, D), lambda i: (i, 0))
    w = pl.BlockSpec((D, OUT), lambda i: (0, 0))
    b = pl.BlockSpec((1, OUT), lambda i: (0, 0))
    o = pl.BlockSpec((blk, OUT), lambda i: (i, 0))
    sh = jax.ShapeDtypeStruct((N, OUT), f32)
    return pl.pallas_call(
        _dense_hk_body, grid=(N // blk,),
        in_specs=[io, io, w, b, w, b],
        out_specs=[o, o],
        out_shape=[sh, sh],
    )(x0, x1, w0, b0, w1, b1)


def _dense_qs_body(x0, x1, w0, b0, w1, b1, g, q0, s0, q1, s1, mq0, mq1):
    i = pl.program_id(0)
    r0 = jnp.dot(x0[...], w0[...], preferred_element_type=f32) + b0[...]
    qa = r0[:, 0:OUT]
    q0[...] = qa
    s0[...] = r0[:, OUT:2 * OUT]
    r1 = jnp.dot(x1[...], w1[...], preferred_element_type=f32) + b1[...]
    qb = r1[:, 0:OUT]
    q1[...] = qb
    s1[...] = r1[:, OUT:2 * OUT]
    neg = jnp.full((1, HEADS), -jnp.inf, f32)
    for ref, x in ((mq0, qa), (mq1, qb)):
        n = jnp.sqrt(jnp.dot(x * x, g[...], preferred_element_type=f32))
        bm = jnp.max(n, axis=0, keepdims=True)
        ref[...] = jnp.maximum(jnp.where(i == 0, neg, ref[...]), bm)


def _dense_qs(x0, x1, w0, b0, w1, b1, g):
    blk = 2000
    io = pl.BlockSpec((blk, D), lambda i: (i, 0))
    w = pl.BlockSpec((D, 2 * OUT), lambda i: (0, 0))
    b = pl.BlockSpec((1, 2 * OUT), lambda i: (0, 0))
    o = pl.BlockSpec((blk, OUT), lambda i: (i, 0))
    shf = jax.ShapeDtypeStruct((N, OUT), f32)
    m = pl.BlockSpec((1, HEADS), lambda i: (0, 0))
    msh = jax.ShapeDtypeStruct((1, HEADS), f32)
    return pl.pallas_call(
        _dense_qs_body, grid=(N // blk,),
        in_specs=[io, io, w, b, w, b,
                  pl.BlockSpec((OUT, HEADS), lambda i: (0, 0))],
        out_specs=[o] * 4 + [m, m],
        out_shape=[shf, shf, shf, shf, msh, msh],
    )(x0, x1, w0, b0, w1, b1, g)


EPW = 4992
NFULL = EPW // CHUNK
TAIL = NFULL * CHUNK * NW


def _sc_hef_body(hk0, hk1, ni0, hi0, ni1, hi1, zeros, out,
                 nvf, nve, hv0, hv1, rows0, rows1, acc,
                 sg0, sg1, sh0, sh1, ss):
    cid = lax.axis_index("c")
    sid = lax.axis_index("s")
    wid = sid * NC + cid
    ebase = pl.multiple_of(wid * EPW, CHUNK)
    nch = jnp.where(wid < 2, NFULL + 1, NFULL)
    hv = (hv0, hv1)
    rows = (rows0, rows1)
    sg = (sg0, sg1)
    sh = (sh0, sh1)
    _per_sub_rows(sid, lambda off, sz: pltpu.sync_copy(
        zeros.at[pl.ds(off, sz)], acc.at[pl.ds(off, sz)]))
    plsc.subcore_barrier()
    for hk, ni, hi in ((hk0, ni0, hi0), (hk1, ni1, hi1)):
        pltpu.async_copy(ni.at[pl.ds(ebase, EPW)], nvf, ss).wait()

        @pl.when(wid < 2)
        def _():
            pltpu.sync_copy(ni.at[pl.ds(pl.multiple_of(TAIL + wid * CHUNK,
                                                       CHUNK), CHUNK)], nve)

        def issue(c, p):
            @pl.when(c < NFULL)
            def _():
                b = pl.multiple_of(c * CHUNK, CHUNK)
                pltpu.async_copy(hi.at[pl.ds(pl.multiple_of(ebase + b, CHUNK),
                                             CHUNK)], hv[p], sh[p])
                pltpu.async_copy(hk.at[nvf.at[pl.ds(b, CHUNK)]], rows[p], sg[p])

            @pl.when(c == NFULL)
            def _():
                off = pl.multiple_of(TAIL + wid * CHUNK, CHUNK)
                pltpu.async_copy(hi.at[pl.ds(off, CHUNK)], hv[p], sh[p])
                pltpu.async_copy(hk.at[nve], rows[p], sg[p])

        def finish(c, p):
            @pl.when(c < nch)
            def _():
                pltpu.make_async_copy(hi.at[pl.ds(0, CHUNK)], hv[p], sh[p]).wait()
                pltpu.make_async_copy(hk.at[nve], rows[p], sg[p]).wait()
                pltpu.sync_copy(rows[p], acc.at[hv[p]], add=True)

        for p in range(2):
            issue(jnp.int32(p), p)

        @pl.loop(0, (NFULL + 2) // 2)
        def _(j):
            for p in range(2):
                c = 2 * j + p
                finish(c, p)

                @pl.when(c + 2 < nch)
                def _():
                    issue(c + 2, p)
    plsc.subcore_barrier()
    _per_sub_rows(sid, lambda off, sz: pltpu.sync_copy(
        acc.at[pl.ds(off, sz)], out.at[cid, pl.ds(off, sz)]))


def _sc_hef(hk0, hk1, ni0, hi0, ni1, hi1, zeros):
    return pl.kernel(
        _sc_hef_body,
        out_type=jax.ShapeDtypeStruct((NC, N, OUT), f32),
        mesh=_mesh(),
        scratch_types=[
            pltpu.VMEM((EPW,), jnp.int32),
            pltpu.VMEM((CHUNK,), jnp.int32),
            pltpu.VMEM((CHUNK,), jnp.int32),
            pltpu.VMEM((CHUNK,), jnp.int32),
            pltpu.VMEM((CHUNK, OUT), f32),
            pltpu.VMEM((CHUNK, OUT), f32),
            pltpu.VMEM_SHARED((N, OUT), f32),
        ] + [pltpu.SemaphoreType.DMA] * 5,
    )(hk0, hk1, ni0, hi0, ni1, hi1, zeros)


def _combine_norms_body(p, g, hef, mh):
    i = pl.program_id(0)
    h = p[0] + p[1]
    hef[...] = h
    n = jnp.sqrt(jnp.dot(h * h, g[...], preferred_element_type=f32))
    bm = jnp.max(n, axis=0, keepdims=True)
    neg = jnp.full((1, HEADS), -jnp.inf, f32)
    mh[...] = jnp.maximum(jnp.where(i == 0, neg, mh[...]), bm)


def _combine_norms(parts, g):
    blk = 2000
    m = pl.BlockSpec((1, HEADS), lambda i: (0, 0))
    msh = jax.ShapeDtypeStruct((1, HEADS), f32)
    return pl.pallas_call(
        _combine_norms_body, grid=(N // blk,),
        in_specs=[pl.BlockSpec((NC, blk, OUT), lambda i: (0, i, 0)),
                  pl.BlockSpec((OUT, HEADS), lambda i: (0, 0))],
        out_specs=[pl.BlockSpec((blk, OUT), lambda i: (i, 0)), m],
        out_shape=[jax.ShapeDtypeStruct((N, OUT), f32), msh],
    )(parts, g)


def _make_sc_qk(ECNT, EPW_, NF, TAIL_, CUT):
    def body(q, hef, ni, hi,
             qg, kg,
             nvf, hvf, nve, hve,
             qr0, qr1, qr2, kr0, kr1, kr2,
             sq0, sq1, sq2, sk0, sk1, sk2,
             wq0, wq1, wq2, wk0, wk1, wk2, ss):
        cid = lax.axis_index("c")
        sid = lax.axis_index("s")
        wid = sid * NC + cid
        ebase = pl.multiple_of(wid * EPW_, CHUNK)
        nch = jnp.where(wid < CUT, NF + 1, NF)
        qr = (qr0, qr1, qr2)
        kr = (kr0, kr1, kr2)
        sq = (sq0, sq1, sq2)
        sk = (sk0, sk1, sk2)
        wq = (wq0, wq1, wq2)
        wk = (wk0, wk1, wk2)
        cpa = pltpu.async_copy(ni.at[pl.ds(ebase, EPW_)], nvf, ss)
        cpb = pltpu.async_copy(hi.at[pl.ds(ebase, EPW_)], hvf, ss)
        cpa.wait()
        cpb.wait()

        @pl.when(wid < CUT)
        def _():
            toff = pl.multiple_of(TAIL_ + wid * CHUNK, CHUNK)
            pltpu.sync_copy(ni.at[pl.ds(toff, CHUNK)], nve)
            pltpu.sync_copy(hi.at[pl.ds(toff, CHUNK)], hve)

        def pre(c, p):
            @pl.when(c < nch)
            def _():
                @pl.when(c >= 3)
                def _():
                    pltpu.make_async_copy(qr[p], qg.at[pl.ds(0, CHUNK)], wq[p]).wait()
                    pltpu.make_async_copy(kr[p], kg.at[pl.ds(0, CHUNK)], wk[p]).wait()

                @pl.when(c < NF)
                def _():
                    b = pl.multiple_of(c * CHUNK, CHUNK)
                    pltpu.async_copy(q.at[nvf.at[pl.ds(b, CHUNK)]], qr[p], sq[p])
                    pltpu.async_copy(hef.at[hvf.at[pl.ds(b, CHUNK)]], kr[p], sk[p])

                @pl.when(c == NF)
                def _():
                    pltpu.async_copy(q.at[nve], qr[p], sq[p])
                    pltpu.async_copy(hef.at[hve], kr[p], sk[p])

        def proc(c, p):
            @pl.when(c < nch)
            def _():
                pltpu.make_async_copy(q.at[nve], qr[p], sq[p]).wait()
                pltpu.make_async_copy(hef.at[hve], kr[p], sk[p]).wait()
                woff = pl.multiple_of(
                    jnp.where(c < NF, ebase + c * CHUNK,
                              TAIL_ + wid * CHUNK), CHUNK)
                pltpu.async_copy(qr[p], qg.at[pl.ds(woff, CHUNK)], wq[p])
                pltpu.async_copy(kr[p], kg.at[pl.ds(woff, CHUNK)], wk[p])

        for p in range(3):
            pre(jnp.int32(p), p)

        @pl.loop(0, (NF + 3) // 3)
        def _(j):
            for p in range(3):
                c = 3 * j + p
                proc(c, p)
                pre(c + 3, p)

        for p in range(3):
            pltpu.make_async_copy(qr[p], qg.at[pl.ds(0, CHUNK)], wq[p]).wait()
            pltpu.make_async_copy(kr[p], kg.at[pl.ds(0, CHUNK)], wk[p]).wait()

    def call(q, hef, ni, hi):
        sh = jax.ShapeDtypeStruct((ECNT, OUT), f32)
        return pl.kernel(
            body,
            out_type=(sh, sh),
            mesh=_mesh(),
            scratch_types=[
                pltpu.VMEM((EPW_,), jnp.int32),
                pltpu.VMEM((EPW_,), jnp.int32),
                pltpu.VMEM((CHUNK,), jnp.int32),
                pltpu.VMEM((CHUNK,), jnp.int32),
                pltpu.VMEM((CHUNK, OUT), f32),
                pltpu.VMEM((CHUNK, OUT), f32),
                pltpu.VMEM((CHUNK, OUT), f32),
                pltpu.VMEM((CHUNK, OUT), f32),
                pltpu.VMEM((CHUNK, OUT), f32),
                pltpu.VMEM((CHUNK, OUT), f32),
            ] + [pltpu.SemaphoreType.DMA] * 13,
        )(q, hef, ni, hi)

    return call


EH = E // 2
EPWH = 2432
TAILH = NW * EPWH
_sc_qk = _make_sc_qk(E, EPW, NFULL, TAIL, 2)
_sc_qk_h = _make_sc_qk(EH, EPWH, EPWH // CHUNK, TAILH, 17)


def _alphaval_body(qg, kg, mq, mh, g, gt, val, se):
    i = pl.program_id(0)
    kgv = kg[...]
    a = jnp.dot(qg[...] * kgv, g[...],
                preferred_element_type=f32) * (1.0 / math.sqrt(DK))
    bound = mq[...] * mh[...] * (1.0 / math.sqrt(DK)) + 1.0
    w = jnp.exp(a - bound)
    w128 = jnp.dot(w, gt[...], preferred_element_type=f32)
    val[...] = kgv * w128
    prev = jnp.where(i == 0, jnp.zeros((1, OUT), f32), se[...])
    se[...] = prev + jnp.sum(w128, axis=0, keepdims=True)


def _alphaval(qg, kg, mq, mh, g, gt):
    blk = 2000
    ecnt = qg.shape[0]
    io = pl.BlockSpec((blk, OUT), lambda i: (i, 0))
    m = pl.BlockSpec((1, HEADS), lambda i: (0, 0))
    return pl.pallas_call(
        _alphaval_body, grid=(ecnt // blk,),
        in_specs=[io, io, m, m,
                  pl.BlockSpec((OUT, HEADS), lambda i: (0, 0)),
                  pl.BlockSpec((HEADS, OUT), lambda i: (0, 0))],
        out_specs=[pl.BlockSpec((blk, OUT), lambda i: (i, 0)),
                   pl.BlockSpec((1, OUT), lambda i: (0, 0))],
        out_shape=[jax.ShapeDtypeStruct((ecnt, OUT), f32),
                   jax.ShapeDtypeStruct((1, OUT), f32)],
    )(qg, kg, mq, mh, g, gt)


def _make_sc_nodeout(slices):
    nsl = len(slices)

    def body(*args):
        refs = args[:2 * nsl]
        zeros = args[2 * nsl]
        out = args[2 * nsl + 1]
        (nv0, nv1, nv2, rv0, rv1, rv2, acc,
         sn0, sn1, sn2, sv0, sv1, sv2) = args[2 * nsl + 2:]
        cid = lax.axis_index("c")
        sid = lax.axis_index("s")
        wid = sid * NC + cid
        nv = (nv0, nv1, nv2)
        rv = (rv0, rv1, rv2)
        sn = (sn0, sn1, sn2)
        sv = (sv0, sv1, sv2)
        _per_sub_rows(sid, lambda off, sz: pltpu.sync_copy(
            zeros.at[pl.ds(off, sz)], acc.at[pl.ds(off, sz)]))
        plsc.subcore_barrier()
        for s, (EPW_, NF, TAIL_, CUT) in enumerate(slices):
            val = refs[s]
            ni = refs[nsl + s]
            ebase = pl.multiple_of(wid * EPW_, CHUNK)
            nch = jnp.where(wid < CUT, NF + 1, NF)

            def issue(c, p):
                off = pl.multiple_of(
                    jnp.where(c < NF, ebase + c * CHUNK,
                              TAIL_ + wid * CHUNK), CHUNK)
                pltpu.async_copy(ni.at[pl.ds(off, CHUNK)], nv[p], sn[p])
                pltpu.async_copy(val.at[pl.ds(off, CHUNK)], rv[p], sv[p])

            def finish(c, p):
                @pl.when(c < nch)
                def _():
                    pltpu.make_async_copy(ni.at[pl.ds(0, CHUNK)], nv[p], sn[p]).wait()
                    pltpu.make_async_copy(val.at[pl.ds(0, CHUNK)], rv[p], sv[p]).wait()
                    pltpu.sync_copy(rv[p], acc.at[nv[p]], add=True)

            for p in range(3):
                issue(jnp.int32(p), p)

            @pl.loop(0, (NF + 3) // 3)
            def _(j):
                for p in range(3):
                    c = 3 * j + p
                    finish(c, p)

                    @pl.when(c + 3 < nch)
                    def _():
                        issue(c + 3, p)
        plsc.subcore_barrier()
        _per_sub_rows(sid, lambda off, sz: pltpu.sync_copy(
            acc.at[pl.ds(off, sz)], out.at[cid, pl.ds(off, sz)]))

    def call(*val_ni):
        vals = val_ni[:nsl]
        nis = val_ni[nsl:2 * nsl]
        zeros = val_ni[2 * nsl]
        return pl.kernel(
            body,
            out_type=jax.ShapeDtypeStruct((NC, N, OUT), f32),
            mesh=_mesh(),
            scratch_types=[
                pltpu.VMEM((CHUNK,), jnp.int32),
                pltpu.VMEM((CHUNK,), jnp.int32),
                pltpu.VMEM((CHUNK,), jnp.int32),
                pltpu.VMEM((CHUNK, OUT), f32),
                pltpu.VMEM((CHUNK, OUT), f32),
                pltpu.VMEM((CHUNK, OUT), f32),
                pltpu.VMEM_SHARED((N, OUT), f32),
            ] + [pltpu.SemaphoreType.DMA] * 6,
        )(*vals, *nis, zeros)

    return call


_sc_nodeout = _make_sc_nodeout([(EPW, NFULL, TAIL, 2)])
_sc_nodeout_pair = _make_sc_nodeout([(EPWH, EPWH // CHUNK, TAILH, 17)] * 2)


def _post_body(parts, sea, seb, skip, aw, ab, g, b, o):
    rec = 1.0 / (sea[...] + seb[...])
    no = (parts[0] + parts[1]) * rec
    merged = jnp.dot(no, aw[...], preferred_element_type=f32) + ab[...]
    y = merged + skip[...]
    mu = jnp.mean(y, axis=-1, keepdims=True)
    var = jnp.mean((y - mu) ** 2, axis=-1, keepdims=True)
    o[...] = (y - mu) * lax.rsqrt(var + 1e-5) * g[...] + b[...]


def _post(parts, sea, seb, skip, aw, ab, g, b):
    blk = 2000
    return pl.pallas_call(
        _post_body, grid=(N // blk,),
        in_specs=[pl.BlockSpec((NC, blk, OUT), lambda i: (0, i, 0)),
                  pl.BlockSpec((1, OUT), lambda i: (0, 0)),
                  pl.BlockSpec((1, OUT), lambda i: (0, 0)),
                  pl.BlockSpec((blk, OUT), lambda i: (i, 0)),
                  pl.BlockSpec((OUT, OUT), lambda i: (0, 0)),
                  pl.BlockSpec((1, OUT), lambda i: (0, 0)),
                  pl.BlockSpec((1, OUT), lambda i: (0, 0)),
                  pl.BlockSpec((1, OUT), lambda i: (0, 0))],
        out_specs=pl.BlockSpec((blk, OUT), lambda i: (i, 0)),
        out_shape=jax.ShapeDtypeStruct((N, OUT), f32),
    )(parts, sea, seb, skip, aw, ab, g, b)


def kernel(x_n0, x_n1, he_index_n0, he_index_n1, max_he_id,
           k_W_n0, k_b_n0, q_W_n0, q_b_n0, a_W_n0, a_b_n0,
           skip_W_n0, skip_b_n0, ln_g_n0, ln_b_n0,
           k_W_n1, k_b_n1, q_W_n1, q_b_n1, a_W_n1, a_b_n1,
           skip_W_n1, skip_b_n1, ln_g_n1, ln_b_n1):
    del max_he_id

    ni0, hi0 = he_index_n0[0], he_index_n0[1]
    ni1, hi1 = he_index_n1[0], he_index_n1[1]

    wqs0 = jnp.concatenate([q_W_n0, skip_W_n0], axis=1)
    bqs0 = jnp.concatenate([q_b_n0, skip_b_n0])[None, :]
    wqs1 = jnp.concatenate([q_W_n1, skip_W_n1], axis=1)
    bqs1 = jnp.concatenate([q_b_n1, skip_b_n1])[None, :]

    eye = jnp.eye(HEADS, dtype=f32)
    g = jnp.repeat(eye, DK, axis=0)
    gt = jnp.repeat(eye, DK, axis=1)
    zeros = jnp.zeros((N, OUT), f32)

    hk0, hk1 = _dense_hk(x_n0, x_n1, k_W_n0, k_b_n0[None, :],
                         k_W_n1, k_b_n1[None, :])
    hef_parts = _sc_hef(hk0, hk1, ni0, hi0, ni1, hi1, zeros)
    q0, s0, q1, s1, mq0, mq1 = _dense_qs(x_n0, x_n1, wqs0, bqs0,
                                         wqs1, bqs1, g)
    hef, mh = _combine_norms(hef_parts, g)

    ni0a, hi0a = ni0[:EH], hi0[:EH]
    ni0b, hi0b = ni0[EH:], hi0[EH:]
    qg0a, kg0a = _sc_qk_h(q0, hef, ni0a, hi0a)
    qg0b, kg0b = _sc_qk_h(q0, hef, ni0b, hi0b)
    qg1, kg1 = _sc_qk(q1, hef, ni1, hi1)
    val0a, se0a = _alphaval(qg0a, kg0a, mq0, mh, g, gt)
    val0b, se0b = _alphaval(qg0b, kg0b, mq0, mh, g, gt)
    val1, se1 = _alphaval(qg1, kg1, mq1, mh, g, gt)

    parts0 = _sc_nodeout_pair(val0a, val0b, ni0a, ni0b, zeros)
    parts1 = _sc_nodeout(val1, ni1, zeros)

    zse = jnp.zeros((1, OUT), f32)
    out0 = _post(parts0, se0a, se0b, s0, a_W_n0, a_b_n0[None, :],
                 ln_g_n0[None, :], ln_b_n0[None, :])
    out1 = _post(parts1, se1, zse, s1, a_W_n1, a_b_n1[None, :],
                 ln_g_n1[None, :], ln_b_n1[None, :])
    return (out0, out1)


# speedup vs baseline: 1.0274x; 1.0274x over previous
import functools
import math

import jax
import jax.numpy as jnp
from jax import lax
from jax.experimental import pallas as pl
from jax.experimental.pallas import tpu as pltpu
from jax.experimental.pallas import tpu_sc as plsc

N = 10000
D = 128
OUT = 128
HEADS = 8
DK = OUT // HEADS
E = 160000
NC = 2
NS = 16
NW = NC * NS
CHUNK = 128
NCHUNK = E // CHUNK
SLAB = 640


def _per_sub_rows(sid, copy_fn):
    @pl.when(sid < NS - 1)
    def _():
        copy_fn(pl.multiple_of(sid * SLAB, SLAB), SLAB)

    @pl.when(sid == NS - 1)
    def _():
        copy_fn((NS - 1) * SLAB, N - (NS - 1) * SLAB)
@functools.cache
def _mesh():
    return plsc.VectorSubcoreMesh(core_axis_name="c", subcore_axis_name="s",
                                  num_cores=NC, num_subcores=NS)

f32 = jnp.float32


def _dense_hk_body(x0, x1, w0, b0, w1, b1, hk0, hk1):
    hk0[...] = jnp.dot(x0[...], w0[...], preferred_element_type=f32) + b0[...]
    hk1[...] = jnp.dot(x1[...], w1[...], preferred_element_type=f32) + b1[...]


def _dense_hk(x0, x1, w0, b0, w1, b1):
    blk = 2000
    io = pl.BlockSpec((blk, D), lambda i: (i, 0))
    w = pl.BlockSpec((D, OUT), lambda i: (0, 0))
    b = pl.BlockSpec((1, OUT), lambda i: (0, 0))
    o = pl.BlockSpec((blk, OUT), lambda i: (i, 0))
    sh = jax.ShapeDtypeStruct((N, OUT), f32)
    return pl.pallas_call(
        _dense_hk_body, grid=(N // blk,),
        in_specs=[io, io, w, b, w, b],
        out_specs=[o, o],
        out_shape=[sh, sh],
    )(x0, x1, w0, b0, w1, b1)


def _dense_qs_body(x0, x1, w0, b0, w1, b1, g, q0, s0, q1, s1, mq0, mq1):
    i = pl.program_id(0)
    r0 = jnp.dot(x0[...], w0[...], preferred_element_type=f32) + b0[...]
    qa = r0[:, 0:OUT]
    q0[...] = qa
    s0[...] = r0[:, OUT:2 * OUT]
    r1 = jnp.dot(x1[...], w1[...], preferred_element_type=f32) + b1[...]
    qb = r1[:, 0:OUT]
    q1[...] = qb
    s1[...] = r1[:, OUT:2 * OUT]
    neg = jnp.full((1, HEADS), -jnp.inf, f32)
    for ref, x in ((mq0, qa), (mq1, qb)):
        n = jnp.sqrt(jnp.dot(x * x, g[...], preferred_element_type=f32))
        bm = jnp.max(n, axis=0, keepdims=True)
        ref[...] = jnp.maximum(jnp.where(i == 0, neg, ref[...]), bm)


def _dense_qs(x0, x1, w0, b0, w1, b1, g):
    blk = 2000
    io = pl.BlockSpec((blk, D), lambda i: (i, 0))
    w = pl.BlockSpec((D, 2 * OUT), lambda i: (0, 0))
    b = pl.BlockSpec((1, 2 * OUT), lambda i: (0, 0))
    o = pl.BlockSpec((blk, OUT), lambda i: (i, 0))
    shf = jax.ShapeDtypeStruct((N, OUT), f32)
    m = pl.BlockSpec((1, HEADS), lambda i: (0, 0))
    msh = jax.ShapeDtypeStruct((1, HEADS), f32)
    return pl.pallas_call(
        _dense_qs_body, grid=(N // blk,),
        in_specs=[io, io, w, b, w, b,
                  pl.BlockSpec((OUT, HEADS), lambda i: (0, 0))],
        out_specs=[o] * 4 + [m, m],
        out_shape=[shf, shf, shf, shf, msh, msh],
    )(x0, x1, w0, b0, w1, b1, g)


EPW = 4992
NFULL = EPW // CHUNK
TAIL = NFULL * CHUNK * NW


def _sc_hef_body(hk0, hk1, ni0, hi0, ni1, hi1, zeros, out,
                 nvf, nve, hv0, hv1, rows0, rows1, acc,
                 sg0, sg1, sh0, sh1, ss):
    cid = lax.axis_index("c")
    sid = lax.axis_index("s")
    wid = sid * NC + cid
    ebase = pl.multiple_of(wid * EPW, CHUNK)
    nch = jnp.where(wid < 2, NFULL + 1, NFULL)
    hv = (hv0, hv1)
    rows = (rows0, rows1)
    sg = (sg0, sg1)
    sh = (sh0, sh1)
    _per_sub_rows(sid, lambda off, sz: pltpu.sync_copy(
        zeros.at[pl.ds(off, sz)], acc.at[pl.ds(off, sz)]))
    plsc.subcore_barrier()
    for hk, ni, hi in ((hk0, ni0, hi0), (hk1, ni1, hi1)):
        pltpu.async_copy(ni.at[pl.ds(ebase, EPW)], nvf, ss).wait()

        @pl.when(wid < 2)
        def _():
            pltpu.sync_copy(ni.at[pl.ds(pl.multiple_of(TAIL + wid * CHUNK,
                                                       CHUNK), CHUNK)], nve)

        def issue(c, p):
            @pl.when(c < NFULL)
            def _():
                b = pl.multiple_of(c * CHUNK, CHUNK)
                pltpu.async_copy(hi.at[pl.ds(pl.multiple_of(ebase + b, CHUNK),
                                             CHUNK)], hv[p], sh[p])
                pltpu.async_copy(hk.at[nvf.at[pl.ds(b, CHUNK)]], rows[p], sg[p])

            @pl.when(c == NFULL)
            def _():
                off = pl.multiple_of(TAIL + wid * CHUNK, CHUNK)
                pltpu.async_copy(hi.at[pl.ds(off, CHUNK)], hv[p], sh[p])
                pltpu.async_copy(hk.at[nve], rows[p], sg[p])

        def finish(c, p):
            @pl.when(c < nch)
            def _():
                pltpu.make_async_copy(hi.at[pl.ds(0, CHUNK)], hv[p], sh[p]).wait()
                pltpu.make_async_copy(hk.at[nve], rows[p], sg[p]).wait()
                pltpu.sync_copy(rows[p], acc.at[hv[p]], add=True)

        for p in range(2):
            issue(jnp.int32(p), p)

        @pl.loop(0, (NFULL + 2) // 2)
        def _(j):
            for p in range(2):
                c = 2 * j + p
                finish(c, p)

                @pl.when(c + 2 < nch)
                def _():
                    issue(c + 2, p)
    plsc.subcore_barrier()
    _per_sub_rows(sid, lambda off, sz: pltpu.sync_copy(
        acc.at[pl.ds(off, sz)], out.at[cid, pl.ds(off, sz)]))


def _sc_hef(hk0, hk1, ni0, hi0, ni1, hi1, zeros):
    return pl.kernel(
        _sc_hef_body,
        out_type=jax.ShapeDtypeStruct((NC, N, OUT), f32),
        mesh=_mesh(),
        scratch_types=[
            pltpu.VMEM((EPW,), jnp.int32),
            pltpu.VMEM((CHUNK,), jnp.int32),
            pltpu.VMEM((CHUNK,), jnp.int32),
            pltpu.VMEM((CHUNK,), jnp.int32),
            pltpu.VMEM((CHUNK, OUT), f32),
            pltpu.VMEM((CHUNK, OUT), f32),
            pltpu.VMEM_SHARED((N, OUT), f32),
        ] + [pltpu.SemaphoreType.DMA] * 5,
    )(hk0, hk1, ni0, hi0, ni1, hi1, zeros)


def _combine_norms_body(p, g, hef, mh):
    i = pl.program_id(0)
    h = p[0] + p[1]
    hef[...] = h
    n = jnp.sqrt(jnp.dot(h * h, g[...], preferred_element_type=f32))
    bm = jnp.max(n, axis=0, keepdims=True)
    neg = jnp.full((1, HEADS), -jnp.inf, f32)
    mh[...] = jnp.maximum(jnp.where(i == 0, neg, mh[...]), bm)


def _combine_norms(parts, g):
    blk = 2000
    m = pl.BlockSpec((1, HEADS), lambda i: (0, 0))
    msh = jax.ShapeDtypeStruct((1, HEADS), f32)
    return pl.pallas_call(
        _combine_norms_body, grid=(N // blk,),
        in_specs=[pl.BlockSpec((NC, blk, OUT), lambda i: (0, i, 0)),
                  pl.BlockSpec((OUT, HEADS), lambda i: (0, 0))],
        out_specs=[pl.BlockSpec((blk, OUT), lambda i: (i, 0)), m],
        out_shape=[jax.ShapeDtypeStruct((N, OUT), f32), msh],
    )(parts, g)


def _make_sc_qk(ECNT, EPW_, NF, TAIL_, CUT):
    def body(q, hef, ni, hi,
             qg, kg,
             nvf, hvf, nve, hve,
             qr0, qr1, qr2, kr0, kr1, kr2,
             sq0, sq1, sq2, sk0, sk1, sk2,
             wq0, wq1, wq2, wk0, wk1, wk2, ss):
        cid = lax.axis_index("c")
        sid = lax.axis_index("s")
        wid = sid * NC + cid
        ebase = pl.multiple_of(wid * EPW_, CHUNK)
        nch = jnp.where(wid < CUT, NF + 1, NF)
        qr = (qr0, qr1, qr2)
        kr = (kr0, kr1, kr2)
        sq = (sq0, sq1, sq2)
        sk = (sk0, sk1, sk2)
        wq = (wq0, wq1, wq2)
        wk = (wk0, wk1, wk2)
        cpa = pltpu.async_copy(ni.at[pl.ds(ebase, EPW_)], nvf, ss)
        cpb = pltpu.async_copy(hi.at[pl.ds(ebase, EPW_)], hvf, ss)
        cpa.wait()
        cpb.wait()

        @pl.when(wid < CUT)
        def _():
            toff = pl.multiple_of(TAIL_ + wid * CHUNK, CHUNK)
            pltpu.sync_copy(ni.at[pl.ds(toff, CHUNK)], nve)
            pltpu.sync_copy(hi.at[pl.ds(toff, CHUNK)], hve)

        def pre(c, p):
            @pl.when(c < nch)
            def _():
                @pl.when(c >= 3)
                def _():
                    pltpu.make_async_copy(qr[p], qg.at[pl.ds(0, CHUNK)], wq[p]).wait()
                    pltpu.make_async_copy(kr[p], kg.at[pl.ds(0, CHUNK)], wk[p]).wait()

                @pl.when(c < NF)
                def _():
                    b = pl.multiple_of(c * CHUNK, CHUNK)
                    pltpu.async_copy(q.at[nvf.at[pl.ds(b, CHUNK)]], qr[p], sq[p])
                    pltpu.async_copy(hef.at[hvf.at[pl.ds(b, CHUNK)]], kr[p], sk[p])

                @pl.when(c == NF)
                def _():
                    pltpu.async_copy(q.at[nve], qr[p], sq[p])
                    pltpu.async_copy(hef.at[hve], kr[p], sk[p])

        def proc(c, p):
            @pl.when(c < nch)
            def _():
                pltpu.make_async_copy(q.at[nve], qr[p], sq[p]).wait()
                pltpu.make_async_copy(hef.at[hve], kr[p], sk[p]).wait()
                woff = pl.multiple_of(
                    jnp.where(c < NF, ebase + c * CHUNK,
                              TAIL_ + wid * CHUNK), CHUNK)
                pltpu.async_copy(qr[p], qg.at[pl.ds(woff, CHUNK)], wq[p])
                pltpu.async_copy(kr[p], kg.at[pl.ds(woff, CHUNK)], wk[p])

        for p in range(3):
            pre(jnp.int32(p), p)

        @pl.loop(0, (NF + 3) // 3)
        def _(j):
            for p in range(3):
                c = 3 * j + p
                proc(c, p)
                pre(c + 3, p)

        for p in range(3):
            pltpu.make_async_copy(qr[p], qg.at[pl.ds(0, CHUNK)], wq[p]).wait()
            pltpu.make_async_copy(kr[p], kg.at[pl.ds(0, CHUNK)], wk[p]).wait()

    def call(q, hef, ni, hi):
        sh = jax.ShapeDtypeStruct((ECNT, OUT), f32)
        return pl.kernel(
            body,
            out_type=(sh, sh),
            mesh=_mesh(),
            scratch_types=[
                pltpu.VMEM((EPW_,), jnp.int32),
                pltpu.VMEM((EPW_,), jnp.int32),
                pltpu.VMEM((CHUNK,), jnp.int32),
                pltpu.VMEM((CHUNK,), jnp.int32),
                pltpu.VMEM((CHUNK, OUT), f32),
                pltpu.VMEM((CHUNK, OUT), f32),
                pltpu.VMEM((CHUNK, OUT), f32),
                pltpu.VMEM((CHUNK, OUT), f32),
                pltpu.VMEM((CHUNK, OUT), f32),
                pltpu.VMEM((CHUNK, OUT), f32),
            ] + [pltpu.SemaphoreType.DMA] * 13,
        )(q, hef, ni, hi)

    return call


EH = E // 2
EPWH = 2432
TAILH = NW * EPWH
_sc_qk = _make_sc_qk(E, EPW, NFULL, TAIL, 2)
_sc_qk_h = _make_sc_qk(EH, EPWH, EPWH // CHUNK, TAILH, 17)


def _alphaval_body(qg, kg, mq, mh, g, gt, val, se):
    i = pl.program_id(0)
    kgv = kg[...]
    a = jnp.dot(qg[...] * kgv, g[...],
                preferred_element_type=f32) * (1.0 / math.sqrt(DK))
    bound = mq[...] * mh[...] * (1.0 / math.sqrt(DK)) + 1.0
    w = jnp.exp(a - bound)
    w128 = jnp.dot(w, gt[...], preferred_element_type=f32)
    val[...] = kgv * w128
    prev = jnp.where(i == 0, jnp.zeros((1, OUT), f32), se[...])
    se[...] = prev + jnp.sum(w128, axis=0, keepdims=True)


def _alphaval(qg, kg, mq, mh, g, gt):
    blk = 5000
    ecnt = qg.shape[0]
    io = pl.BlockSpec((blk, OUT), lambda i: (i, 0))
    m = pl.BlockSpec((1, HEADS), lambda i: (0, 0))
    return pl.pallas_call(
        _alphaval_body, grid=(ecnt // blk,),
        in_specs=[io, io, m, m,
                  pl.BlockSpec((OUT, HEADS), lambda i: (0, 0)),
                  pl.BlockSpec((HEADS, OUT), lambda i: (0, 0))],
        out_specs=[pl.BlockSpec((blk, OUT), lambda i: (i, 0)),
                   pl.BlockSpec((1, OUT), lambda i: (0, 0))],
        out_shape=[jax.ShapeDtypeStruct((ecnt, OUT), f32),
                   jax.ShapeDtypeStruct((1, OUT), f32)],
    )(qg, kg, mq, mh, g, gt)


def _make_sc_nodeout(slices):
    nsl = len(slices)

    def body(*args):
        refs = args[:2 * nsl]
        zeros = args[2 * nsl]
        out = args[2 * nsl + 1]
        (nv0, nv1, nv2, rv0, rv1, rv2, acc,
         sn0, sn1, sn2, sv0, sv1, sv2) = args[2 * nsl + 2:]
        cid = lax.axis_index("c")
        sid = lax.axis_index("s")
        wid = sid * NC + cid
        nv = (nv0, nv1, nv2)
        rv = (rv0, rv1, rv2)
        sn = (sn0, sn1, sn2)
        sv = (sv0, sv1, sv2)
        _per_sub_rows(sid, lambda off, sz: pltpu.sync_copy(
            zeros.at[pl.ds(off, sz)], acc.at[pl.ds(off, sz)]))
        plsc.subcore_barrier()
        for s, (EPW_, NF, TAIL_, CUT) in enumerate(slices):
            val = refs[s]
            ni = refs[nsl + s]
            ebase = pl.multiple_of(wid * EPW_, CHUNK)
            nch = jnp.where(wid < CUT, NF + 1, NF)

            def issue(c, p):
                off = pl.multiple_of(
                    jnp.where(c < NF, ebase + c * CHUNK,
                              TAIL_ + wid * CHUNK), CHUNK)
                pltpu.async_copy(ni.at[pl.ds(off, CHUNK)], nv[p], sn[p])
                pltpu.async_copy(val.at[pl.ds(off, CHUNK)], rv[p], sv[p])

            def finish(c, p):
                @pl.when(c < nch)
                def _():
                    pltpu.make_async_copy(ni.at[pl.ds(0, CHUNK)], nv[p], sn[p]).wait()
                    pltpu.make_async_copy(val.at[pl.ds(0, CHUNK)], rv[p], sv[p]).wait()
                    pltpu.sync_copy(rv[p], acc.at[nv[p]], add=True)

            for p in range(3):
                issue(jnp.int32(p), p)

            @pl.loop(0, (NF + 3) // 3)
            def _(j):
                for p in range(3):
                    c = 3 * j + p
                    finish(c, p)

                    @pl.when(c + 3 < nch)
                    def _():
                        issue(c + 3, p)
        plsc.subcore_barrier()
        _per_sub_rows(sid, lambda off, sz: pltpu.sync_copy(
            acc.at[pl.ds(off, sz)], out.at[cid, pl.ds(off, sz)]))

    def call(*val_ni):
        vals = val_ni[:nsl]
        nis = val_ni[nsl:2 * nsl]
        zeros = val_ni[2 * nsl]
        return pl.kernel(
            body,
            out_type=jax.ShapeDtypeStruct((NC, N, OUT), f32),
            mesh=_mesh(),
            scratch_types=[
                pltpu.VMEM((CHUNK,), jnp.int32),
                pltpu.VMEM((CHUNK,), jnp.int32),
                pltpu.VMEM((CHUNK,), jnp.int32),
                pltpu.VMEM((CHUNK, OUT), f32),
                pltpu.VMEM((CHUNK, OUT), f32),
                pltpu.VMEM((CHUNK, OUT), f32),
                pltpu.VMEM_SHARED((N, OUT), f32),
            ] + [pltpu.SemaphoreType.DMA] * 6,
        )(*vals, *nis, zeros)

    return call


_sc_nodeout = _make_sc_nodeout([(EPW, NFULL, TAIL, 2)])
_sc_nodeout_pair = _make_sc_nodeout([(EPWH, EPWH // CHUNK, TAILH, 17)] * 2)


def _post_body(parts, sea, seb, skip, aw, ab, g, b, o):
    rec = 1.0 / (sea[...] + seb[...])
    no = (parts[0] + parts[1]) * rec
    merged = jnp.dot(no, aw[...], preferred_element_type=f32) + ab[...]
    y = merged + skip[...]
    mu = jnp.mean(y, axis=-1, keepdims=True)
    var = jnp.mean((y - mu) ** 2, axis=-1, keepdims=True)
    o[...] = (y - mu) * lax.rsqrt(var + 1e-5) * g[...] + b[...]


def _post(parts, sea, seb, skip, aw, ab, g, b):
    blk = 2000
    return pl.pallas_call(
        _post_body, grid=(N // blk,),
        in_specs=[pl.BlockSpec((NC, blk, OUT), lambda i: (0, i, 0)),
                  pl.BlockSpec((1, OUT), lambda i: (0, 0)),
                  pl.BlockSpec((1, OUT), lambda i: (0, 0)),
                  pl.BlockSpec((blk, OUT), lambda i: (i, 0)),
                  pl.BlockSpec((OUT, OUT), lambda i: (0, 0)),
                  pl.BlockSpec((1, OUT), lambda i: (0, 0)),
                  pl.BlockSpec((1, OUT), lambda i: (0, 0)),
                  pl.BlockSpec((1, OUT), lambda i: (0, 0))],
        out_specs=pl.BlockSpec((blk, OUT), lambda i: (i, 0)),
        out_shape=jax.ShapeDtypeStruct((N, OUT), f32),
    )(parts, sea, seb, skip, aw, ab, g, b)


def kernel(x_n0, x_n1, he_index_n0, he_index_n1, max_he_id,
           k_W_n0, k_b_n0, q_W_n0, q_b_n0, a_W_n0, a_b_n0,
           skip_W_n0, skip_b_n0, ln_g_n0, ln_b_n0,
           k_W_n1, k_b_n1, q_W_n1, q_b_n1, a_W_n1, a_b_n1,
           skip_W_n1, skip_b_n1, ln_g_n1, ln_b_n1):
    del max_he_id

    ni0, hi0 = he_index_n0[0], he_index_n0[1]
    ni1, hi1 = he_index_n1[0], he_index_n1[1]

    wqs0 = jnp.concatenate([q_W_n0, skip_W_n0], axis=1)
    bqs0 = jnp.concatenate([q_b_n0, skip_b_n0])[None, :]
    wqs1 = jnp.concatenate([q_W_n1, skip_W_n1], axis=1)
    bqs1 = jnp.concatenate([q_b_n1, skip_b_n1])[None, :]

    eye = jnp.eye(HEADS, dtype=f32)
    g = jnp.repeat(eye, DK, axis=0)
    gt = jnp.repeat(eye, DK, axis=1)
    zeros = jnp.zeros((N, OUT), f32)

    hk0, hk1 = _dense_hk(x_n0, x_n1, k_W_n0, k_b_n0[None, :],
                         k_W_n1, k_b_n1[None, :])
    hef_parts = _sc_hef(hk0, hk1, ni0, hi0, ni1, hi1, zeros)
    q0, s0, q1, s1, mq0, mq1 = _dense_qs(x_n0, x_n1, wqs0, bqs0,
                                         wqs1, bqs1, g)
    hef, mh = _combine_norms(hef_parts, g)

    ni0a, hi0a = ni0[:EH], hi0[:EH]
    ni0b, hi0b = ni0[EH:], hi0[EH:]
    qg0a, kg0a = _sc_qk_h(q0, hef, ni0a, hi0a)
    qg0b, kg0b = _sc_qk_h(q0, hef, ni0b, hi0b)
    qg1, kg1 = _sc_qk(q1, hef, ni1, hi1)
    val0a, se0a = _alphaval(qg0a, kg0a, mq0, mh, g, gt)
    val0b, se0b = _alphaval(qg0b, kg0b, mq0, mh, g, gt)
    val1, se1 = _alphaval(qg1, kg1, mq1, mh, g, gt)

    parts0 = _sc_nodeout_pair(val0a, val0b, ni0a, ni0b, zeros)
    parts1 = _sc_nodeout(val1, ni1, zeros)

    zse = jnp.zeros((1, OUT), f32)
    out0 = _post(parts0, se0a, se0b, s0, a_W_n0, a_b_n0[None, :],
                 ln_g_n0[None, :], ln_b_n0[None, :])
    out1 = _post(parts1, se1, zse, s1, a_W_n1, a_b_n1[None, :],
                 ln_g_n1[None, :], ln_b_n1[None, :])
    return (out0, out1)


# speedup vs baseline: 1.0385x; 1.0108x over previous
import functools
import math

import jax
import jax.numpy as jnp
from jax import lax
from jax.experimental import pallas as pl
from jax.experimental.pallas import tpu as pltpu
from jax.experimental.pallas import tpu_sc as plsc

N = 10000
D = 128
OUT = 128
HEADS = 8
DK = OUT // HEADS
E = 160000
NC = 2
NS = 16
NW = NC * NS
CHUNK = 128
NCHUNK = E // CHUNK
SLAB = 640


def _per_sub_rows(sid, copy_fn):
    @pl.when(sid < NS - 1)
    def _():
        copy_fn(pl.multiple_of(sid * SLAB, SLAB), SLAB)

    @pl.when(sid == NS - 1)
    def _():
        copy_fn((NS - 1) * SLAB, N - (NS - 1) * SLAB)
@functools.cache
def _mesh():
    return plsc.VectorSubcoreMesh(core_axis_name="c", subcore_axis_name="s",
                                  num_cores=NC, num_subcores=NS)

f32 = jnp.float32


def _dense_hk_body(x0, x1, w0, b0, w1, b1, hk0, hk1):
    hk0[...] = jnp.dot(x0[...], w0[...], preferred_element_type=f32) + b0[...]
    hk1[...] = jnp.dot(x1[...], w1[...], preferred_element_type=f32) + b1[...]


def _dense_hk(x0, x1, w0, b0, w1, b1):
    blk = 5000
    io = pl.BlockSpec((blk, D), lambda i: (i, 0))
    w = pl.BlockSpec((D, OUT), lambda i: (0, 0))
    b = pl.BlockSpec((1, OUT), lambda i: (0, 0))
    o = pl.BlockSpec((blk, OUT), lambda i: (i, 0))
    sh = jax.ShapeDtypeStruct((N, OUT), f32)
    return pl.pallas_call(
        _dense_hk_body, grid=(N // blk,),
        in_specs=[io, io, w, b, w, b],
        out_specs=[o, o],
        out_shape=[sh, sh],
    )(x0, x1, w0, b0, w1, b1)


def _dense_qs_body(x0, x1, w0, b0, w1, b1, g, q0, s0, q1, s1, mq0, mq1):
    i = pl.program_id(0)
    r0 = jnp.dot(x0[...], w0[...], preferred_element_type=f32) + b0[...]
    qa = r0[:, 0:OUT]
    q0[...] = qa
    s0[...] = r0[:, OUT:2 * OUT]
    r1 = jnp.dot(x1[...], w1[...], preferred_element_type=f32) + b1[...]
    qb = r1[:, 0:OUT]
    q1[...] = qb
    s1[...] = r1[:, OUT:2 * OUT]
    neg = jnp.full((1, HEADS), -jnp.inf, f32)
    for ref, x in ((mq0, qa), (mq1, qb)):
        n = jnp.sqrt(jnp.dot(x * x, g[...], preferred_element_type=f32))
        bm = jnp.max(n, axis=0, keepdims=True)
        ref[...] = jnp.maximum(jnp.where(i == 0, neg, ref[...]), bm)


def _dense_qs(x0, x1, w0, b0, w1, b1, g):
    blk = 5000
    io = pl.BlockSpec((blk, D), lambda i: (i, 0))
    w = pl.BlockSpec((D, 2 * OUT), lambda i: (0, 0))
    b = pl.BlockSpec((1, 2 * OUT), lambda i: (0, 0))
    o = pl.BlockSpec((blk, OUT), lambda i: (i, 0))
    shf = jax.ShapeDtypeStruct((N, OUT), f32)
    m = pl.BlockSpec((1, HEADS), lambda i: (0, 0))
    msh = jax.ShapeDtypeStruct((1, HEADS), f32)
    return pl.pallas_call(
        _dense_qs_body, grid=(N // blk,),
        in_specs=[io, io, w, b, w, b,
                  pl.BlockSpec((OUT, HEADS), lambda i: (0, 0))],
        out_specs=[o] * 4 + [m, m],
        out_shape=[shf, shf, shf, shf, msh, msh],
    )(x0, x1, w0, b0, w1, b1, g)


EPW = 4992
NFULL = EPW // CHUNK
TAIL = NFULL * CHUNK * NW


def _sc_hef_body(hk0, hk1, ni0, hi0, ni1, hi1, zeros, out,
                 nvf, nve, hv0, hv1, rows0, rows1, acc,
                 sg0, sg1, sh0, sh1, ss):
    cid = lax.axis_index("c")
    sid = lax.axis_index("s")
    wid = sid * NC + cid
    ebase = pl.multiple_of(wid * EPW, CHUNK)
    nch = jnp.where(wid < 2, NFULL + 1, NFULL)
    hv = (hv0, hv1)
    rows = (rows0, rows1)
    sg = (sg0, sg1)
    sh = (sh0, sh1)
    _per_sub_rows(sid, lambda off, sz: pltpu.sync_copy(
        zeros.at[pl.ds(off, sz)], acc.at[pl.ds(off, sz)]))
    plsc.subcore_barrier()
    for hk, ni, hi in ((hk0, ni0, hi0), (hk1, ni1, hi1)):
        pltpu.async_copy(ni.at[pl.ds(ebase, EPW)], nvf, ss).wait()

        @pl.when(wid < 2)
        def _():
            pltpu.sync_copy(ni.at[pl.ds(pl.multiple_of(TAIL + wid * CHUNK,
                                                       CHUNK), CHUNK)], nve)

        def issue(c, p):
            @pl.when(c < NFULL)
            def _():
                b = pl.multiple_of(c * CHUNK, CHUNK)
                pltpu.async_copy(hi.at[pl.ds(pl.multiple_of(ebase + b, CHUNK),
                                             CHUNK)], hv[p], sh[p])
                pltpu.async_copy(hk.at[nvf.at[pl.ds(b, CHUNK)]], rows[p], sg[p])

            @pl.when(c == NFULL)
            def _():
                off = pl.multiple_of(TAIL + wid * CHUNK, CHUNK)
                pltpu.async_copy(hi.at[pl.ds(off, CHUNK)], hv[p], sh[p])
                pltpu.async_copy(hk.at[nve], rows[p], sg[p])

        def finish(c, p):
            @pl.when(c < nch)
            def _():
                pltpu.make_async_copy(hi.at[pl.ds(0, CHUNK)], hv[p], sh[p]).wait()
                pltpu.make_async_copy(hk.at[nve], rows[p], sg[p]).wait()
                pltpu.sync_copy(rows[p], acc.at[hv[p]], add=True)

        for p in range(2):
            issue(jnp.int32(p), p)

        @pl.loop(0, (NFULL + 2) // 2)
        def _(j):
            for p in range(2):
                c = 2 * j + p
                finish(c, p)

                @pl.when(c + 2 < nch)
                def _():
                    issue(c + 2, p)
    plsc.subcore_barrier()
    _per_sub_rows(sid, lambda off, sz: pltpu.sync_copy(
        acc.at[pl.ds(off, sz)], out.at[cid, pl.ds(off, sz)]))


def _sc_hef(hk0, hk1, ni0, hi0, ni1, hi1, zeros):
    return pl.kernel(
        _sc_hef_body,
        out_type=jax.ShapeDtypeStruct((NC, N, OUT), f32),
        mesh=_mesh(),
        scratch_types=[
            pltpu.VMEM((EPW,), jnp.int32),
            pltpu.VMEM((CHUNK,), jnp.int32),
            pltpu.VMEM((CHUNK,), jnp.int32),
            pltpu.VMEM((CHUNK,), jnp.int32),
            pltpu.VMEM((CHUNK, OUT), f32),
            pltpu.VMEM((CHUNK, OUT), f32),
            pltpu.VMEM_SHARED((N, OUT), f32),
        ] + [pltpu.SemaphoreType.DMA] * 5,
    )(hk0, hk1, ni0, hi0, ni1, hi1, zeros)


def _combine_norms_body(p, g, hef, mh):
    i = pl.program_id(0)
    h = p[0] + p[1]
    hef[...] = h
    n = jnp.sqrt(jnp.dot(h * h, g[...], preferred_element_type=f32))
    bm = jnp.max(n, axis=0, keepdims=True)
    neg = jnp.full((1, HEADS), -jnp.inf, f32)
    mh[...] = jnp.maximum(jnp.where(i == 0, neg, mh[...]), bm)


def _combine_norms(parts, g):
    blk = 5000
    m = pl.BlockSpec((1, HEADS), lambda i: (0, 0))
    msh = jax.ShapeDtypeStruct((1, HEADS), f32)
    return pl.pallas_call(
        _combine_norms_body, grid=(N // blk,),
        in_specs=[pl.BlockSpec((NC, blk, OUT), lambda i: (0, i, 0)),
                  pl.BlockSpec((OUT, HEADS), lambda i: (0, 0))],
        out_specs=[pl.BlockSpec((blk, OUT), lambda i: (i, 0)), m],
        out_shape=[jax.ShapeDtypeStruct((N, OUT), f32), msh],
    )(parts, g)


def _make_sc_qk(ECNT, EPW_, NF, TAIL_, CUT):
    def body(q, hef, ni, hi,
             qg, kg,
             nvf, hvf, nve, hve,
             qr0, qr1, qr2, kr0, kr1, kr2,
             sq0, sq1, sq2, sk0, sk1, sk2,
             wq0, wq1, wq2, wk0, wk1, wk2, ss):
        cid = lax.axis_index("c")
        sid = lax.axis_index("s")
        wid = sid * NC + cid
        ebase = pl.multiple_of(wid * EPW_, CHUNK)
        nch = jnp.where(wid < CUT, NF + 1, NF)
        qr = (qr0, qr1, qr2)
        kr = (kr0, kr1, kr2)
        sq = (sq0, sq1, sq2)
        sk = (sk0, sk1, sk2)
        wq = (wq0, wq1, wq2)
        wk = (wk0, wk1, wk2)
        cpa = pltpu.async_copy(ni.at[pl.ds(ebase, EPW_)], nvf, ss)
        cpb = pltpu.async_copy(hi.at[pl.ds(ebase, EPW_)], hvf, ss)
        cpa.wait()
        cpb.wait()

        @pl.when(wid < CUT)
        def _():
            toff = pl.multiple_of(TAIL_ + wid * CHUNK, CHUNK)
            pltpu.sync_copy(ni.at[pl.ds(toff, CHUNK)], nve)
            pltpu.sync_copy(hi.at[pl.ds(toff, CHUNK)], hve)

        def pre(c, p):
            @pl.when(c < nch)
            def _():
                @pl.when(c >= 3)
                def _():
                    pltpu.make_async_copy(qr[p], qg.at[pl.ds(0, CHUNK)], wq[p]).wait()
                    pltpu.make_async_copy(kr[p], kg.at[pl.ds(0, CHUNK)], wk[p]).wait()

                @pl.when(c < NF)
                def _():
                    b = pl.multiple_of(c * CHUNK, CHUNK)
                    pltpu.async_copy(q.at[nvf.at[pl.ds(b, CHUNK)]], qr[p], sq[p])
                    pltpu.async_copy(hef.at[hvf.at[pl.ds(b, CHUNK)]], kr[p], sk[p])

                @pl.when(c == NF)
                def _():
                    pltpu.async_copy(q.at[nve], qr[p], sq[p])
                    pltpu.async_copy(hef.at[hve], kr[p], sk[p])

        def proc(c, p):
            @pl.when(c < nch)
            def _():
                pltpu.make_async_copy(q.at[nve], qr[p], sq[p]).wait()
                pltpu.make_async_copy(hef.at[hve], kr[p], sk[p]).wait()
                woff = pl.multiple_of(
                    jnp.where(c < NF, ebase + c * CHUNK,
                              TAIL_ + wid * CHUNK), CHUNK)
                pltpu.async_copy(qr[p], qg.at[pl.ds(woff, CHUNK)], wq[p])
                pltpu.async_copy(kr[p], kg.at[pl.ds(woff, CHUNK)], wk[p])

        for p in range(3):
            pre(jnp.int32(p), p)

        @pl.loop(0, (NF + 3) // 3)
        def _(j):
            for p in range(3):
                c = 3 * j + p
                proc(c, p)
                pre(c + 3, p)

        for p in range(3):
            pltpu.make_async_copy(qr[p], qg.at[pl.ds(0, CHUNK)], wq[p]).wait()
            pltpu.make_async_copy(kr[p], kg.at[pl.ds(0, CHUNK)], wk[p]).wait()

    def call(q, hef, ni, hi):
        sh = jax.ShapeDtypeStruct((ECNT, OUT), f32)
        return pl.kernel(
            body,
            out_type=(sh, sh),
            mesh=_mesh(),
            scratch_types=[
                pltpu.VMEM((EPW_,), jnp.int32),
                pltpu.VMEM((EPW_,), jnp.int32),
                pltpu.VMEM((CHUNK,), jnp.int32),
                pltpu.VMEM((CHUNK,), jnp.int32),
                pltpu.VMEM((CHUNK, OUT), f32),
                pltpu.VMEM((CHUNK, OUT), f32),
                pltpu.VMEM((CHUNK, OUT), f32),
                pltpu.VMEM((CHUNK, OUT), f32),
                pltpu.VMEM((CHUNK, OUT), f32),
                pltpu.VMEM((CHUNK, OUT), f32),
            ] + [pltpu.SemaphoreType.DMA] * 13,
        )(q, hef, ni, hi)

    return call


EH = E // 2
EPWH = 2432
TAILH = NW * EPWH
_sc_qk = _make_sc_qk(E, EPW, NFULL, TAIL, 2)
_sc_qk_h = _make_sc_qk(EH, EPWH, EPWH // CHUNK, TAILH, 17)


def _alphaval_body(qg, kg, mq, mh, g, gt, val, se):
    i = pl.program_id(0)
    kgv = kg[...]
    a = jnp.dot(qg[...] * kgv, g[...],
                preferred_element_type=f32) * (1.0 / math.sqrt(DK))
    bound = mq[...] * mh[...] * (1.0 / math.sqrt(DK)) + 1.0
    w = jnp.exp(a - bound)
    w128 = jnp.dot(w, gt[...], preferred_element_type=f32)
    val[...] = kgv * w128
    prev = jnp.where(i == 0, jnp.zeros((1, OUT), f32), se[...])
    se[...] = prev + jnp.sum(w128, axis=0, keepdims=True)


def _alphaval(qg, kg, mq, mh, g, gt):
    blk = 5000
    ecnt = qg.shape[0]
    io = pl.BlockSpec((blk, OUT), lambda i: (i, 0))
    m = pl.BlockSpec((1, HEADS), lambda i: (0, 0))
    return pl.pallas_call(
        _alphaval_body, grid=(ecnt // blk,),
        in_specs=[io, io, m, m,
                  pl.BlockSpec((OUT, HEADS), lambda i: (0, 0)),
                  pl.BlockSpec((HEADS, OUT), lambda i: (0, 0))],
        out_specs=[pl.BlockSpec((blk, OUT), lambda i: (i, 0)),
                   pl.BlockSpec((1, OUT), lambda i: (0, 0))],
        out_shape=[jax.ShapeDtypeStruct((ecnt, OUT), f32),
                   jax.ShapeDtypeStruct((1, OUT), f32)],
    )(qg, kg, mq, mh, g, gt)


def _make_sc_nodeout(slices):
    nsl = len(slices)

    def body(*args):
        refs = args[:2 * nsl]
        zeros = args[2 * nsl]
        out = args[2 * nsl + 1]
        (nv0, nv1, nv2, rv0, rv1, rv2, acc,
         sn0, sn1, sn2, sv0, sv1, sv2) = args[2 * nsl + 2:]
        cid = lax.axis_index("c")
        sid = lax.axis_index("s")
        wid = sid * NC + cid
        nv = (nv0, nv1, nv2)
        rv = (rv0, rv1, rv2)
        sn = (sn0, sn1, sn2)
        sv = (sv0, sv1, sv2)
        _per_sub_rows(sid, lambda off, sz: pltpu.sync_copy(
            zeros.at[pl.ds(off, sz)], acc.at[pl.ds(off, sz)]))
        plsc.subcore_barrier()
        for s, (EPW_, NF, TAIL_, CUT) in enumerate(slices):
            val = refs[s]
            ni = refs[nsl + s]
            ebase = pl.multiple_of(wid * EPW_, CHUNK)
            nch = jnp.where(wid < CUT, NF + 1, NF)

            def issue(c, p):
                off = pl.multiple_of(
                    jnp.where(c < NF, ebase + c * CHUNK,
                              TAIL_ + wid * CHUNK), CHUNK)
                pltpu.async_copy(ni.at[pl.ds(off, CHUNK)], nv[p], sn[p])
                pltpu.async_copy(val.at[pl.ds(off, CHUNK)], rv[p], sv[p])

            def finish(c, p):
                @pl.when(c < nch)
                def _():
                    pltpu.make_async_copy(ni.at[pl.ds(0, CHUNK)], nv[p], sn[p]).wait()
                    pltpu.make_async_copy(val.at[pl.ds(0, CHUNK)], rv[p], sv[p]).wait()
                    pltpu.sync_copy(rv[p], acc.at[nv[p]], add=True)

            for p in range(3):
                issue(jnp.int32(p), p)

            @pl.loop(0, (NF + 3) // 3)
            def _(j):
                for p in range(3):
                    c = 3 * j + p
                    finish(c, p)

                    @pl.when(c + 3 < nch)
                    def _():
                        issue(c + 3, p)
        plsc.subcore_barrier()
        _per_sub_rows(sid, lambda off, sz: pltpu.sync_copy(
            acc.at[pl.ds(off, sz)], out.at[cid, pl.ds(off, sz)]))

    def call(*val_ni):
        vals = val_ni[:nsl]
        nis = val_ni[nsl:2 * nsl]
        zeros = val_ni[2 * nsl]
        return pl.kernel(
            body,
            out_type=jax.ShapeDtypeStruct((NC, N, OUT), f32),
            mesh=_mesh(),
            scratch_types=[
                pltpu.VMEM((CHUNK,), jnp.int32),
                pltpu.VMEM((CHUNK,), jnp.int32),
                pltpu.VMEM((CHUNK,), jnp.int32),
                pltpu.VMEM((CHUNK, OUT), f32),
                pltpu.VMEM((CHUNK, OUT), f32),
                pltpu.VMEM((CHUNK, OUT), f32),
                pltpu.VMEM_SHARED((N, OUT), f32),
            ] + [pltpu.SemaphoreType.DMA] * 6,
        )(*vals, *nis, zeros)

    return call


_sc_nodeout = _make_sc_nodeout([(EPW, NFULL, TAIL, 2)])
_sc_nodeout_pair = _make_sc_nodeout([(EPWH, EPWH // CHUNK, TAILH, 17)] * 2)


def _post_body(parts, sea, seb, skip, aw, ab, g, b, o):
    rec = 1.0 / (sea[...] + seb[...])
    no = (parts[0] + parts[1]) * rec
    merged = jnp.dot(no, aw[...], preferred_element_type=f32) + ab[...]
    y = merged + skip[...]
    mu = jnp.mean(y, axis=-1, keepdims=True)
    var = jnp.mean((y - mu) ** 2, axis=-1, keepdims=True)
    o[...] = (y - mu) * lax.rsqrt(var + 1e-5) * g[...] + b[...]


def _post(parts, sea, seb, skip, aw, ab, g, b):
    blk = 5000
    return pl.pallas_call(
        _post_body, grid=(N // blk,),
        in_specs=[pl.BlockSpec((NC, blk, OUT), lambda i: (0, i, 0)),
                  pl.BlockSpec((1, OUT), lambda i: (0, 0)),
                  pl.BlockSpec((1, OUT), lambda i: (0, 0)),
                  pl.BlockSpec((blk, OUT), lambda i: (i, 0)),
                  pl.BlockSpec((OUT, OUT), lambda i: (0, 0)),
                  pl.BlockSpec((1, OUT), lambda i: (0, 0)),
                  pl.BlockSpec((1, OUT), lambda i: (0, 0)),
                  pl.BlockSpec((1, OUT), lambda i: (0, 0))],
        out_specs=pl.BlockSpec((blk, OUT), lambda i: (i, 0)),
        out_shape=jax.ShapeDtypeStruct((N, OUT), f32),
    )(parts, sea, seb, skip, aw, ab, g, b)


def kernel(x_n0, x_n1, he_index_n0, he_index_n1, max_he_id,
           k_W_n0, k_b_n0, q_W_n0, q_b_n0, a_W_n0, a_b_n0,
           skip_W_n0, skip_b_n0, ln_g_n0, ln_b_n0,
           k_W_n1, k_b_n1, q_W_n1, q_b_n1, a_W_n1, a_b_n1,
           skip_W_n1, skip_b_n1, ln_g_n1, ln_b_n1):
    del max_he_id

    ni0, hi0 = he_index_n0[0], he_index_n0[1]
    ni1, hi1 = he_index_n1[0], he_index_n1[1]

    wqs0 = jnp.concatenate([q_W_n0, skip_W_n0], axis=1)
    bqs0 = jnp.concatenate([q_b_n0, skip_b_n0])[None, :]
    wqs1 = jnp.concatenate([q_W_n1, skip_W_n1], axis=1)
    bqs1 = jnp.concatenate([q_b_n1, skip_b_n1])[None, :]

    eye = jnp.eye(HEADS, dtype=f32)
    g = jnp.repeat(eye, DK, axis=0)
    gt = jnp.repeat(eye, DK, axis=1)
    zeros = jnp.zeros((N, OUT), f32)

    hk0, hk1 = _dense_hk(x_n0, x_n1, k_W_n0, k_b_n0[None, :],
                         k_W_n1, k_b_n1[None, :])
    hef_parts = _sc_hef(hk0, hk1, ni0, hi0, ni1, hi1, zeros)
    q0, s0, q1, s1, mq0, mq1 = _dense_qs(x_n0, x_n1, wqs0, bqs0,
                                         wqs1, bqs1, g)
    hef, mh = _combine_norms(hef_parts, g)

    ni0a, hi0a = ni0[:EH], hi0[:EH]
    ni0b, hi0b = ni0[EH:], hi0[EH:]
    qg0a, kg0a = _sc_qk_h(q0, hef, ni0a, hi0a)
    qg0b, kg0b = _sc_qk_h(q0, hef, ni0b, hi0b)
    qg1, kg1 = _sc_qk(q1, hef, ni1, hi1)
    val0a, se0a = _alphaval(qg0a, kg0a, mq0, mh, g, gt)
    val0b, se0b = _alphaval(qg0b, kg0b, mq0, mh, g, gt)
    val1, se1 = _alphaval(qg1, kg1, mq1, mh, g, gt)

    parts0 = _sc_nodeout_pair(val0a, val0b, ni0a, ni0b, zeros)
    parts1 = _sc_nodeout(val1, ni1, zeros)

    zse = jnp.zeros((1, OUT), f32)
    out0 = _post(parts0, se0a, se0b, s0, a_W_n0, a_b_n0[None, :],
                 ln_g_n0[None, :], ln_b_n0[None, :])
    out1 = _post(parts1, se1, zse, s1, a_W_n1, a_b_n1[None, :],
                 ln_g_n1[None, :], ln_b_n1[None, :])
    return (out0, out1)
